# baseline (speedup 1.0000x reference)
import functools

import jax
import jax.numpy as jnp
from jax import lax
from jax.experimental import pallas as pl
from jax.experimental.pallas import tpu as pltpu
from jax.experimental.pallas import tpu_sc as plsc

B = 16384
D = 32
V = 1000000
NC = 2
NS = 16
NW = NC * NS
BPW = B // NW
VBLK = 32768
VL = 16


def _tc_scores_body(ut_ref, it_ref, w_ref, b_ref, su_ref, si_ref):
    su_ref[...] = jnp.exp(-(jnp.sum(ut_ref[...] * w_ref[0:D, :], axis=0) + b_ref[0]))
    si_ref[...] = jnp.exp(-jnp.sum(it_ref[...] * w_ref[D:, :], axis=0))


def _tc_scores(ut_t, it_t, w_t, fc_b):
    grid = (V + VBLK - 1) // VBLK
    return pl.pallas_call(
        _tc_scores_body,
        grid=(grid,),
        in_specs=[
            pl.BlockSpec((D, VBLK), lambda i: (0, i)),
            pl.BlockSpec((D, VBLK), lambda i: (0, i)),
            pl.BlockSpec((2 * D, 1), lambda i: (0, 0)),
            pl.BlockSpec(memory_space=pltpu.SMEM),
        ],
        out_specs=(
            pl.BlockSpec((VBLK,), lambda i: (i,)),
            pl.BlockSpec((VBLK,), lambda i: (i,)),
        ),
        out_shape=(
            jax.ShapeDtypeStruct((V,), jnp.float32),
            jax.ShapeDtypeStruct((V,), jnp.float32),
        ),
    )(ut_t, it_t, w_t, fc_b)


def _sc_gather_head(user_ids, item_ids, score_u, score_i):
    mesh = plsc.VectorSubcoreMesh(core_axis_name="c", subcore_axis_name="s")

    @functools.partial(
        pl.kernel,
        mesh=mesh,
        out_type=jax.ShapeDtypeStruct((B,), jnp.float32),
        scratch_types=[
            pltpu.VMEM((BPW,), jnp.int32),
            pltpu.VMEM((BPW,), jnp.int32),
            pltpu.VMEM((BPW,), jnp.float32),
            pltpu.VMEM((BPW,), jnp.float32),
            pltpu.VMEM((BPW,), jnp.float32),
            pltpu.SemaphoreType.DMA,
            pltpu.SemaphoreType.DMA,
        ],
    )
    def k(uid_hbm, iid_hbm, su_hbm, si_hbm, o_hbm,
          uidx_v, iidx_v, su_v, si_v, o_v, sem_u, sem_i):
        wid = lax.axis_index("s") * NC + lax.axis_index("c")
        base = wid * BPW
        pltpu.sync_copy(uid_hbm.at[pl.ds(base, BPW)], uidx_v)
        pltpu.sync_copy(iid_hbm.at[pl.ds(base, BPW)], iidx_v)
        cu = pltpu.async_copy(su_hbm.at[uidx_v], su_v, sem_u)
        ci = pltpu.async_copy(si_hbm.at[iidx_v], si_v, sem_i)
        cu.wait()
        ci.wait()

        @pl.loop(0, BPW, step=VL)
        def _(j):
            t = su_v[pl.ds(j, VL)] * si_v[pl.ds(j, VL)]
            o_v[pl.ds(j, VL)] = 1.0 / (1.0 + t)

        pltpu.sync_copy(o_v, o_hbm.at[pl.ds(base, BPW)])

    return k(user_ids, item_ids, score_u, score_i)


def kernel(user_ids, item_ids, user_table, item_table, fc_w, fc_b):
    ut_t = user_table.T
    it_t = item_table.T
    score_u, score_i = _tc_scores(ut_t, it_t, fc_w.T, fc_b)
    out = _sc_gather_head(user_ids.astype(jnp.int32), item_ids.astype(jnp.int32),
                          score_u, score_i)
    return out.reshape(B, 1)


# speedup vs baseline: 1.0471x; 1.0471x over previous
import functools

import jax
import jax.numpy as jnp
from jax import lax
from jax.experimental import pallas as pl
from jax.experimental.pallas import tpu as pltpu
from jax.experimental.pallas import tpu_sc as plsc

B = 16384
D = 32
V = 1000000
NC = 2
NS = 16
NW = NC * NS
BPW = B // NW
VBLK = 49152
VL = 16


def _tc_scores_body(ut_ref, it_ref, w_ref, b_ref, su_ref, si_ref):
    su_ref[...] = jnp.sum(ut_ref[...] * w_ref[0:D, :], axis=0) + b_ref[0]
    si_ref[...] = jnp.sum(it_ref[...] * w_ref[D:, :], axis=0)


def _tc_scores(ut_t, it_t, w_t, fc_b):
    grid = (V + VBLK - 1) // VBLK
    return pl.pallas_call(
        _tc_scores_body,
        grid=(grid,),
        in_specs=[
            pl.BlockSpec((D, VBLK), lambda i: (0, i)),
            pl.BlockSpec((D, VBLK), lambda i: (0, i)),
            pl.BlockSpec((2 * D, 1), lambda i: (0, 0)),
            pl.BlockSpec(memory_space=pltpu.SMEM),
        ],
        out_specs=(
            pl.BlockSpec((VBLK,), lambda i: (i,)),
            pl.BlockSpec((VBLK,), lambda i: (i,)),
        ),
        out_shape=(
            jax.ShapeDtypeStruct((V,), jnp.float32),
            jax.ShapeDtypeStruct((V,), jnp.float32),
        ),
    )(ut_t, it_t, w_t, fc_b)


def _sc_gather_head(user_ids, item_ids, score_u, score_i):
    mesh = plsc.VectorSubcoreMesh(core_axis_name="c", subcore_axis_name="s")

    @functools.partial(
        pl.kernel,
        mesh=mesh,
        out_type=jax.ShapeDtypeStruct((B,), jnp.float32),
        scratch_types=[
            pltpu.VMEM((BPW,), jnp.int32),
            pltpu.VMEM((BPW,), jnp.int32),
            pltpu.VMEM((BPW,), jnp.float32),
            pltpu.VMEM((BPW,), jnp.float32),
            pltpu.VMEM((BPW,), jnp.float32),
            pltpu.SemaphoreType.DMA,
            pltpu.SemaphoreType.DMA,
        ],
    )
    def k(uid_hbm, iid_hbm, su_hbm, si_hbm, o_hbm,
          uidx_v, iidx_v, su_v, si_v, o_v, sem_u, sem_i):
        wid = lax.axis_index("s") * NC + lax.axis_index("c")
        base = wid * BPW
        pltpu.sync_copy(uid_hbm.at[pl.ds(base, BPW)], uidx_v)
        pltpu.sync_copy(iid_hbm.at[pl.ds(base, BPW)], iidx_v)
        cu = pltpu.async_copy(su_hbm.at[uidx_v], su_v, sem_u)
        ci = pltpu.async_copy(si_hbm.at[iidx_v], si_v, sem_i)
        cu.wait()
        ci.wait()

        @pl.loop(0, BPW, step=VL)
        def _(j):
            t = su_v[pl.ds(j, VL)] + si_v[pl.ds(j, VL)]
            o_v[pl.ds(j, VL)] = 1.0 / (1.0 + jnp.exp(-t))

        pltpu.sync_copy(o_v, o_hbm.at[pl.ds(base, BPW)])

    return k(user_ids, item_ids, score_u, score_i)


def kernel(user_ids, item_ids, user_table, item_table, fc_w, fc_b):
    ut_t = user_table.T
    it_t = item_table.T
    score_u, score_i = _tc_scores(ut_t, it_t, fc_w.T, fc_b)
    out = _sc_gather_head(user_ids.astype(jnp.int32), item_ids.astype(jnp.int32),
                          score_u, score_i)
    return out.reshape(B, 1)
